# baseline (speedup 1.0000x reference)
import functools

import jax
import jax.numpy as jnp
from jax.experimental import pallas as pl
from jax.experimental.pallas import tpu as pltpu


def _chamfer_kernel(p_ref, g_ref, out_ref,
                    row_best, row_bestex, col_best, col_bestex, sums,
                    *, nb, ni, nj, denom_m, denom_n):
    b = pl.program_id(0)
    i = pl.program_id(1)
    j = pl.program_id(2)

    p = p_ref[0]
    g = g_ref[0]

    px, py, pz = p[:, 0:1], p[:, 1:2], p[:, 2:3]
    gx, gy, gz = g[0:1, :], g[1:2, :], g[2:3, :]

    aa = px * px + py * py + pz * pz
    bb = gx * gx + gy * gy + gz * gz
    t = aa + bb

    dims = (((1,), (0,)), ((), ()))
    ab_lo = jax.lax.dot_general(
        p.astype(jnp.bfloat16), g.astype(jnp.bfloat16), dims,
        preferred_element_type=jnp.float32)
    approx = t - 2.0 * ab_lo
    ab_hi = jax.lax.dot_general(p, g, dims,
                                preferred_element_type=jnp.float32,
                                precision=jax.lax.Precision.HIGHEST)
    exact = jnp.maximum(t - 2.0 * ab_hi, 0.0)

    inf = jnp.float32(jnp.inf)

    tile_min = jnp.min(approx, axis=1, keepdims=True)
    tile_ex = jnp.min(jnp.where(approx == tile_min, exact, inf),
                      axis=1, keepdims=True)
    prev_min = jnp.where(j == 0, inf, row_best[...])
    prev_ex = jnp.where(j == 0, inf, row_bestex[...])
    upd = tile_min < prev_min
    row_best[...] = jnp.where(upd, tile_min, prev_min)
    row_bestex[...] = jnp.where(upd, tile_ex, prev_ex)

    bn = approx.shape[1]
    csl = (slice(None), pl.ds(j * bn, bn))
    ctile_min = jnp.min(approx, axis=0, keepdims=True)
    ctile_ex = jnp.min(jnp.where(approx == ctile_min, exact, inf),
                       axis=0, keepdims=True)
    cprev_min = jnp.where(i == 0, inf, col_best[csl])
    cprev_ex = jnp.where(i == 0, inf, col_bestex[csl])
    cupd = ctile_min < cprev_min
    col_best[csl] = jnp.where(cupd, ctile_min, cprev_min)
    col_bestex[csl] = jnp.where(cupd, ctile_ex, cprev_ex)

    @pl.when(j == nj - 1)
    def _():
        s = jnp.sum(jnp.sqrt(row_bestex[...] + 1e-8))
        prev = jnp.where((b == 0) & (i == 0), 0.0, sums[0])
        sums[0] = prev + s

    @pl.when(i == ni - 1)
    def _():
        s = jnp.sum(jnp.sqrt(col_bestex[csl] + 1e-8))
        prev = jnp.where((b == 0) & (j == 0), 0.0, sums[1])
        sums[1] = prev + s

    @pl.when((b == nb - 1) & (i == ni - 1) & (j == nj - 1))
    def _():
        loss = sums[0] / denom_m + sums[1] / denom_n
        out_ref[...] = jnp.full((1, 1), loss, jnp.float32)


def _chamfer_call(p_t, gt_pc, bm, bn):
    B, M, _ = p_t.shape
    N = gt_pc.shape[2]
    ni = M // bm
    nj = N // bn
    return pl.pallas_call(
        functools.partial(_chamfer_kernel, nb=B, ni=ni, nj=nj,
                          denom_m=float(B * M), denom_n=float(B * N)),
        grid=(B, ni, nj),
        in_specs=[
            pl.BlockSpec((1, bm, 3), lambda b, i, j: (b, i, 0)),
            pl.BlockSpec((1, 3, bn), lambda b, i, j: (b, 0, j)),
        ],
        out_specs=pl.BlockSpec((1, 1), lambda b, i, j: (0, 0)),
        out_shape=jax.ShapeDtypeStruct((1, 1), jnp.float32),
        scratch_shapes=[
            pltpu.VMEM((bm, 1), jnp.float32),
            pltpu.VMEM((bm, 1), jnp.float32),
            pltpu.VMEM((1, N), jnp.float32),
            pltpu.VMEM((1, N), jnp.float32),
            pltpu.SMEM((2,), jnp.float32),
        ],
    )(p_t, gt_pc)


@jax.jit
def kernel(predict_pc, gt_pc):
    B, _, M = predict_pc.shape
    N = gt_pc.shape[2]
    bm = min(512, M)
    bn = min(1024, N)
    p_t = jnp.swapaxes(predict_pc, 1, 2)
    out = _chamfer_call(p_t, gt_pc, bm, bn)
    return out[0, 0]


# speedup vs baseline: 1.6845x; 1.6845x over previous
import functools

import jax
import jax.numpy as jnp
from jax.experimental import pallas as pl
from jax.experimental.pallas import tpu as pltpu


def _chamfer_kernel(p_ref, g_ref, out_ref,
                    row_best, row_bestex, col_best, col_bestex, sums,
                    *, nb, ni, nj, denom_m, denom_n):
    b = pl.program_id(0)
    i = pl.program_id(1)
    j = pl.program_id(2)

    p = p_ref[0]
    g = g_ref[0]

    px, py, pz = p[:, 0:1], p[:, 1:2], p[:, 2:3]
    gx, gy, gz = g[0:1, :], g[1:2, :], g[2:3, :]

    aa = px * px + py * py + pz * pz
    bb = gx * gx + gy * gy + gz * gz
    t = aa + bb

    dims = (((1,), (0,)), ((), ()))
    ab2 = jax.lax.dot_general(
        p.astype(jnp.bfloat16), g.astype(jnp.bfloat16) * jnp.bfloat16(2.0),
        dims, preferred_element_type=jnp.float32)
    approx = t - ab2
    dx = px - gx
    dy = py - gy
    dz = pz - gz
    exact = dx * dx + dy * dy + dz * dz

    inf = jnp.float32(jnp.inf)

    tile_min = jnp.min(approx, axis=1, keepdims=True)
    tile_ex = jnp.min(jnp.where(approx == tile_min, exact, inf),
                      axis=1, keepdims=True)
    prev_min = jnp.where(j == 0, inf, row_best[...])
    prev_ex = jnp.where(j == 0, inf, row_bestex[...])
    upd = tile_min < prev_min
    row_best[...] = jnp.where(upd, tile_min, prev_min)
    row_bestex[...] = jnp.where(upd, tile_ex, prev_ex)

    bn = approx.shape[1]
    csl = (slice(None), pl.ds(j * bn, bn))
    ctile_min = jnp.min(approx, axis=0, keepdims=True)
    ctile_ex = jnp.min(jnp.where(approx == ctile_min, exact, inf),
                       axis=0, keepdims=True)
    cprev_min = jnp.where(i == 0, inf, col_best[csl])
    cprev_ex = jnp.where(i == 0, inf, col_bestex[csl])
    cupd = ctile_min < cprev_min
    col_best[csl] = jnp.where(cupd, ctile_min, cprev_min)
    col_bestex[csl] = jnp.where(cupd, ctile_ex, cprev_ex)

    @pl.when(j == nj - 1)
    def _():
        s = jnp.sum(jnp.sqrt(row_bestex[...] + 1e-8))
        prev = jnp.where((b == 0) & (i == 0), 0.0, sums[0])
        sums[0] = prev + s

    @pl.when(i == ni - 1)
    def _():
        s = jnp.sum(jnp.sqrt(col_bestex[csl] + 1e-8))
        prev = jnp.where((b == 0) & (j == 0), 0.0, sums[1])
        sums[1] = prev + s

    @pl.when((b == nb - 1) & (i == ni - 1) & (j == nj - 1))
    def _():
        loss = sums[0] / denom_m + sums[1] / denom_n
        out_ref[...] = jnp.full((1, 1), loss, jnp.float32)


def _chamfer_call(p_t, gt_pc, bm, bn):
    B, M, _ = p_t.shape
    N = gt_pc.shape[2]
    ni = M // bm
    nj = N // bn
    return pl.pallas_call(
        functools.partial(_chamfer_kernel, nb=B, ni=ni, nj=nj,
                          denom_m=float(B * M), denom_n=float(B * N)),
        grid=(B, ni, nj),
        in_specs=[
            pl.BlockSpec((1, bm, 3), lambda b, i, j: (b, i, 0)),
            pl.BlockSpec((1, 3, bn), lambda b, i, j: (b, 0, j)),
        ],
        out_specs=pl.BlockSpec((1, 1), lambda b, i, j: (0, 0)),
        out_shape=jax.ShapeDtypeStruct((1, 1), jnp.float32),
        scratch_shapes=[
            pltpu.VMEM((bm, 1), jnp.float32),
            pltpu.VMEM((bm, 1), jnp.float32),
            pltpu.VMEM((1, N), jnp.float32),
            pltpu.VMEM((1, N), jnp.float32),
            pltpu.SMEM((2,), jnp.float32),
        ],
    )(p_t, gt_pc)


@jax.jit
def kernel(predict_pc, gt_pc):
    B, _, M = predict_pc.shape
    N = gt_pc.shape[2]
    bm = min(512, M)
    bn = min(2048, N)
    p_t = jnp.swapaxes(predict_pc, 1, 2)
    out = _chamfer_call(p_t, gt_pc, bm, bn)
    return out[0, 0]


# speedup vs baseline: 2.1775x; 1.2927x over previous
import functools

import jax
import jax.numpy as jnp
from jax import lax
from jax.experimental import pallas as pl
from jax.experimental.pallas import tpu as pltpu
from jax.experimental.pallas import tpu_sc as plsc


def _argmin_kernel(p_ref, g_ref, rowidx_ref, colidx_ref,
                   row_best, row_idx, col_best, col_idx, *, ni, nj):
    i = pl.program_id(1)
    j = pl.program_id(2)

    p = p_ref[0]
    g = g_ref[0]
    bm = p.shape[0]
    bn = g.shape[1]

    px, py, pz = p[:, 0:1], p[:, 1:2], p[:, 2:3]
    gx, gy, gz = g[0:1, :], g[1:2, :], g[2:3, :]
    aa = px * px + py * py + pz * pz
    bb = gx * gx + gy * gy + gz * gz
    t = aa + bb

    ab2 = jax.lax.dot_general(
        p.astype(jnp.bfloat16), g.astype(jnp.bfloat16) * jnp.bfloat16(2.0),
        (((1,), (0,)), ((), ())), preferred_element_type=jnp.float32)
    approx = t - ab2

    inf = jnp.float32(jnp.inf)
    big = jnp.int32(0x3FFFFFFF)

    tmin = jnp.min(approx, axis=1, keepdims=True)
    li = lax.broadcasted_iota(jnp.int32, (bm, bn), 1)
    tidx = jnp.min(jnp.where(approx == tmin, li, big),
                   axis=1, keepdims=True) + j * bn
    pmin = jnp.where(j == 0, inf, row_best[...])
    upd = tmin < pmin
    row_best[...] = jnp.where(upd, tmin, pmin)
    row_idx[...] = jnp.where(upd, tidx, row_idx[...])

    csl = (slice(None), pl.ds(j * bn, bn))
    ctmin = jnp.min(approx, axis=0, keepdims=True)
    si = lax.broadcasted_iota(jnp.int32, (bm, bn), 0)
    ctidx = jnp.min(jnp.where(approx == ctmin, si, big),
                    axis=0, keepdims=True) + i * bm
    cpmin = jnp.where(i == 0, inf, col_best[csl])
    cupd = ctmin < cpmin
    col_best[csl] = jnp.where(cupd, ctmin, cpmin)
    col_idx[csl] = jnp.where(cupd, ctidx, col_idx[csl])

    @pl.when(j == nj - 1)
    def _():
        rowidx_ref[0] = row_idx[...]

    @pl.when(i == ni - 1)
    def _():
        colidx_ref[0] = col_idx[csl]


def _argmin_call(p_t, gt_pc, bm, bn):
    B, M, _ = p_t.shape
    N = gt_pc.shape[2]
    ni = M // bm
    nj = N // bn
    return pl.pallas_call(
        functools.partial(_argmin_kernel, ni=ni, nj=nj),
        grid=(B, ni, nj),
        in_specs=[
            pl.BlockSpec((1, bm, 3), lambda b, i, j: (b, i, 0)),
            pl.BlockSpec((1, 3, bn), lambda b, i, j: (b, 0, j)),
        ],
        out_specs=[
            pl.BlockSpec((1, bm, 1), lambda b, i, j: (b, i, 0)),
            pl.BlockSpec((1, 1, bn), lambda b, i, j: (b, 0, j)),
        ],
        out_shape=[
            jax.ShapeDtypeStruct((B, M, 1), jnp.int32),
            jax.ShapeDtypeStruct((B, 1, N), jnp.int32),
        ],
        scratch_shapes=[
            pltpu.VMEM((bm, 1), jnp.float32),
            pltpu.VMEM((bm, 1), jnp.int32),
            pltpu.VMEM((1, N), jnp.float32),
            pltpu.VMEM((1, N), jnp.int32),
        ],
    )(p_t, gt_pc)


_NC, _NS = 2, 16
_NW = _NC * _NS
_LANES = 16


def _sc_gather_kernel(pred_h, gt_h, ir_h, ic_h, d2f_h, d2b_h,
                      tx_v, ty_v, tz_v, idx_v, qx_v, qy_v, qz_v, o_v,
                      *, B, M, N):
    wid = lax.axis_index("s") * _NC + lax.axis_index("c")
    wpb = _NW // B
    batch = wid // wpb
    chunk_f = M // wpb
    chunk_b = N // wpb
    qoff_f = (wid % wpb) * chunk_f
    qoff_b = (wid % wpb) * chunk_b

    def one_direction(table_h, tsize, query_h, qsize, qoff, idx_h, out_h,
                      out_base, chunk):
        pltpu.sync_copy(table_h.at[pl.ds((batch * 3 + 0) * tsize, tsize)], tx_v)
        pltpu.sync_copy(table_h.at[pl.ds((batch * 3 + 1) * tsize, tsize)], ty_v)
        pltpu.sync_copy(table_h.at[pl.ds((batch * 3 + 2) * tsize, tsize)], tz_v)
        pltpu.sync_copy(query_h.at[pl.ds((batch * 3 + 0) * qsize + qoff, chunk)],
                        qx_v.at[pl.ds(0, chunk)])
        pltpu.sync_copy(query_h.at[pl.ds((batch * 3 + 1) * qsize + qoff, chunk)],
                        qy_v.at[pl.ds(0, chunk)])
        pltpu.sync_copy(query_h.at[pl.ds((batch * 3 + 2) * qsize + qoff, chunk)],
                        qz_v.at[pl.ds(0, chunk)])
        pltpu.sync_copy(idx_h.at[pl.ds(out_base, chunk)],
                        idx_v.at[pl.ds(0, chunk)])
        for q in range(chunk // _LANES):
            sl = pl.ds(q * _LANES, _LANES)
            iv = idx_v[sl]
            dx = qx_v[sl] - plsc.load_gather(tx_v, [iv])
            dy = qy_v[sl] - plsc.load_gather(ty_v, [iv])
            dz = qz_v[sl] - plsc.load_gather(tz_v, [iv])
            o_v[sl] = dx * dx + dy * dy + dz * dz
        pltpu.sync_copy(o_v.at[pl.ds(0, chunk)], out_h.at[pl.ds(out_base, chunk)])

    one_direction(gt_h, N, pred_h, M, qoff_f, ir_h, d2f_h,
                  batch * M + qoff_f, chunk_f)
    one_direction(pred_h, M, gt_h, N, qoff_b, ic_h, d2b_h,
                  batch * N + qoff_b, chunk_b)


def _sc_gather_call(pred_flat, gt_flat, ir, ic, B, M, N):
    tmax = max(M, N)
    cmax = max(M, N) // (_NW // B)
    return pl.kernel(
        functools.partial(_sc_gather_kernel, B=B, M=M, N=N),
        out_type=[jax.ShapeDtypeStruct((B * M,), jnp.float32),
                  jax.ShapeDtypeStruct((B * N,), jnp.float32)],
        mesh=plsc.VectorSubcoreMesh(core_axis_name="c", subcore_axis_name="s"),
        compiler_params=pltpu.CompilerParams(needs_layout_passes=False),
        scratch_types=[
            pltpu.VMEM((tmax,), jnp.float32),
            pltpu.VMEM((tmax,), jnp.float32),
            pltpu.VMEM((tmax,), jnp.float32),
            pltpu.VMEM((cmax,), jnp.int32),
            pltpu.VMEM((cmax,), jnp.float32),
            pltpu.VMEM((cmax,), jnp.float32),
            pltpu.VMEM((cmax,), jnp.float32),
            pltpu.VMEM((cmax,), jnp.float32),
        ],
    )(pred_flat, gt_flat, ir, ic)


def _reduce_kernel(d2f_ref, d2b_ref, out_ref, *, denom_m, denom_n):
    s_f = jnp.sum(jnp.sqrt(d2f_ref[...] + 1e-8))
    s_b = jnp.sum(jnp.sqrt(d2b_ref[...] + 1e-8))
    out_ref[...] = jnp.full((1, 1), s_f / denom_m + s_b / denom_n, jnp.float32)


def _reduce_call(d2f, d2b, denom_m, denom_n):
    rows_f = d2f.size // 128
    rows_b = d2b.size // 128
    return pl.pallas_call(
        functools.partial(_reduce_kernel, denom_m=denom_m, denom_n=denom_n),
        out_shape=jax.ShapeDtypeStruct((1, 1), jnp.float32),
    )(d2f.reshape(rows_f, 128), d2b.reshape(rows_b, 128))


@jax.jit
def kernel(predict_pc, gt_pc):
    B, _, M = predict_pc.shape
    N = gt_pc.shape[2]
    bm = min(512, M)
    bn = min(2048, N)
    p_t = jnp.swapaxes(predict_pc, 1, 2)
    idx_row, idx_col = _argmin_call(p_t, gt_pc, bm, bn)

    d2f, d2b = _sc_gather_call(
        predict_pc.reshape(B * 3 * M), gt_pc.reshape(B * 3 * N),
        idx_row.reshape(B * M), idx_col.reshape(B * N), B, M, N)

    out = _reduce_call(d2f, d2b, float(B * M), float(B * N))
    return out[0, 0]


# speedup vs baseline: 2.4088x; 1.1062x over previous
import functools

import jax
import jax.numpy as jnp
from jax import lax
from jax.experimental import pallas as pl
from jax.experimental.pallas import tpu as pltpu
from jax.experimental.pallas import tpu_sc as plsc


def _argmin_kernel(p_ref, g_ref, rowidx_ref, colidx_ref,
                   row_best, row_idx, col_best, col_idx, *, ni, nj):
    i = pl.program_id(1)
    j = pl.program_id(2)

    p = p_ref[0]
    g = g_ref[0]
    bm = p.shape[0]
    bn = g.shape[1]

    px, py, pz = p[:, 0:1], p[:, 1:2], p[:, 2:3]
    gx, gy, gz = g[0:1, :], g[1:2, :], g[2:3, :]
    aa = px * px + py * py + pz * pz
    bb = gx * gx + gy * gy + gz * gz
    t = aa + bb

    ab2 = jax.lax.dot_general(
        p.astype(jnp.bfloat16), g.astype(jnp.bfloat16) * jnp.bfloat16(2.0),
        (((1,), (0,)), ((), ())), preferred_element_type=jnp.float32)
    approx = t - ab2

    inf = jnp.float32(jnp.inf)
    big = jnp.int32(0x3FFFFFFF)

    tmin = jnp.min(approx, axis=1, keepdims=True)
    li = lax.broadcasted_iota(jnp.int32, (bm, bn), 1)
    tidx = jnp.min(jnp.where(approx == tmin, li, big),
                   axis=1, keepdims=True) + j * bn
    pmin = jnp.where(j == 0, inf, row_best[...])
    upd = tmin < pmin
    row_best[...] = jnp.where(upd, tmin, pmin)
    row_idx[...] = jnp.where(upd, tidx, row_idx[...])

    csl = (slice(None), pl.ds(j * bn, bn))
    ctmin = jnp.min(approx, axis=0, keepdims=True)
    si = lax.broadcasted_iota(jnp.int32, (bm, bn), 0)
    ctidx = jnp.min(jnp.where(approx == ctmin, si, big),
                    axis=0, keepdims=True) + i * bm
    cpmin = jnp.where(i == 0, inf, col_best[csl])
    cupd = ctmin < cpmin
    col_best[csl] = jnp.where(cupd, ctmin, cpmin)
    col_idx[csl] = jnp.where(cupd, ctidx, col_idx[csl])

    @pl.when(j == nj - 1)
    def _():
        rowidx_ref[0] = row_idx[...]

    @pl.when(i == ni - 1)
    def _():
        colidx_ref[0] = col_idx[csl]


def _argmin_call(p_t, gt_pc, bm, bn):
    B, M, _ = p_t.shape
    N = gt_pc.shape[2]
    ni = M // bm
    nj = N // bn
    return pl.pallas_call(
        functools.partial(_argmin_kernel, ni=ni, nj=nj),
        grid=(B, ni, nj),
        in_specs=[
            pl.BlockSpec((1, bm, 3), lambda b, i, j: (b, i, 0)),
            pl.BlockSpec((1, 3, bn), lambda b, i, j: (b, 0, j)),
        ],
        out_specs=[
            pl.BlockSpec((1, bm, 1), lambda b, i, j: (b, i, 0)),
            pl.BlockSpec((1, 1, bn), lambda b, i, j: (b, 0, j)),
        ],
        out_shape=[
            jax.ShapeDtypeStruct((B, M, 1), jnp.int32),
            jax.ShapeDtypeStruct((B, 1, N), jnp.int32),
        ],
        scratch_shapes=[
            pltpu.VMEM((bm, 1), jnp.float32),
            pltpu.VMEM((bm, 1), jnp.int32),
            pltpu.VMEM((1, N), jnp.float32),
            pltpu.VMEM((1, N), jnp.int32),
        ],
    )(p_t, gt_pc)


_NC, _NS = 2, 16
_NW = _NC * _NS
_LANES = 16


def _sc_gather_kernel(pred_h, gt_h, ir_h, ic_h, d2f_h, d2b_h,
                      tx_v, ty_v, tz_v, idx_v, qx_v, qy_v, qz_v, o_v,
                      *, B, M, N):
    wid = lax.axis_index("s") * _NC + lax.axis_index("c")
    wpb = _NW // B
    batch = wid // wpb
    chunk_f = M // wpb
    chunk_b = N // wpb
    qoff_f = (wid % wpb) * chunk_f
    qoff_b = (wid % wpb) * chunk_b

    def one_direction(table_h, tsize, query_h, qsize, qoff, idx_h, out_h,
                      out_base, chunk):
        pltpu.sync_copy(table_h.at[pl.ds((batch * 3 + 0) * tsize, tsize)], tx_v)
        pltpu.sync_copy(table_h.at[pl.ds((batch * 3 + 1) * tsize, tsize)], ty_v)
        pltpu.sync_copy(table_h.at[pl.ds((batch * 3 + 2) * tsize, tsize)], tz_v)
        pltpu.sync_copy(query_h.at[pl.ds((batch * 3 + 0) * qsize + qoff, chunk)],
                        qx_v.at[pl.ds(0, chunk)])
        pltpu.sync_copy(query_h.at[pl.ds((batch * 3 + 1) * qsize + qoff, chunk)],
                        qy_v.at[pl.ds(0, chunk)])
        pltpu.sync_copy(query_h.at[pl.ds((batch * 3 + 2) * qsize + qoff, chunk)],
                        qz_v.at[pl.ds(0, chunk)])
        pltpu.sync_copy(idx_h.at[pl.ds(out_base, chunk)],
                        idx_v.at[pl.ds(0, chunk)])
        for q in range(chunk // _LANES):
            sl = pl.ds(q * _LANES, _LANES)
            iv = idx_v[sl]
            dx = qx_v[sl] - plsc.load_gather(tx_v, [iv])
            dy = qy_v[sl] - plsc.load_gather(ty_v, [iv])
            dz = qz_v[sl] - plsc.load_gather(tz_v, [iv])
            o_v[sl] = dx * dx + dy * dy + dz * dz
        pltpu.sync_copy(o_v.at[pl.ds(0, chunk)], out_h.at[pl.ds(out_base, chunk)])

    one_direction(gt_h, N, pred_h, M, qoff_f, ir_h, d2f_h,
                  batch * M + qoff_f, chunk_f)
    one_direction(pred_h, M, gt_h, N, qoff_b, ic_h, d2b_h,
                  batch * N + qoff_b, chunk_b)


def _sc_gather_call(pred_flat, gt_flat, ir, ic, B, M, N):
    tmax = max(M, N)
    cmax = max(M, N) // (_NW // B)
    return pl.kernel(
        functools.partial(_sc_gather_kernel, B=B, M=M, N=N),
        out_type=[jax.ShapeDtypeStruct((B * M,), jnp.float32),
                  jax.ShapeDtypeStruct((B * N,), jnp.float32)],
        mesh=plsc.VectorSubcoreMesh(core_axis_name="c", subcore_axis_name="s"),
        compiler_params=pltpu.CompilerParams(needs_layout_passes=False),
        scratch_types=[
            pltpu.VMEM((tmax,), jnp.float32),
            pltpu.VMEM((tmax,), jnp.float32),
            pltpu.VMEM((tmax,), jnp.float32),
            pltpu.VMEM((cmax,), jnp.int32),
            pltpu.VMEM((cmax,), jnp.float32),
            pltpu.VMEM((cmax,), jnp.float32),
            pltpu.VMEM((cmax,), jnp.float32),
            pltpu.VMEM((cmax,), jnp.float32),
        ],
    )(pred_flat, gt_flat, ir, ic)


def _reduce_kernel(d2f_ref, d2b_ref, out_ref, *, denom_m, denom_n):
    s_f = jnp.sum(jnp.sqrt(d2f_ref[...] + 1e-8))
    s_b = jnp.sum(jnp.sqrt(d2b_ref[...] + 1e-8))
    out_ref[...] = jnp.full((1, 1), s_f / denom_m + s_b / denom_n, jnp.float32)


def _reduce_call(d2f, d2b, denom_m, denom_n):
    rows_f = d2f.size // 128
    rows_b = d2b.size // 128
    return pl.pallas_call(
        functools.partial(_reduce_kernel, denom_m=denom_m, denom_n=denom_n),
        out_shape=jax.ShapeDtypeStruct((1, 1), jnp.float32),
    )(d2f.reshape(rows_f, 128), d2b.reshape(rows_b, 128))


@jax.jit
def kernel(predict_pc, gt_pc):
    B, _, M = predict_pc.shape
    N = gt_pc.shape[2]
    bm = min(512, M)
    bn = min(4096, N)
    p_t = jnp.swapaxes(predict_pc, 1, 2)
    idx_row, idx_col = _argmin_call(p_t, gt_pc, bm, bn)

    d2f, d2b = _sc_gather_call(
        predict_pc.reshape(B * 3 * M), gt_pc.reshape(B * 3 * N),
        idx_row.reshape(B * M), idx_col.reshape(B * N), B, M, N)

    out = _reduce_call(d2f, d2b, float(B * M), float(B * N))
    return out[0, 0]


# speedup vs baseline: 2.5223x; 1.0471x over previous
import functools

import jax
import jax.numpy as jnp
from jax import lax
from jax.experimental import pallas as pl
from jax.experimental.pallas import tpu as pltpu
from jax.experimental.pallas import tpu_sc as plsc


def _argmin_kernel(p_ref, g_ref, rowidx_ref, colidx_ref,
                   row_best, row_idx, col_best, col_idx, *, ni, nj):
    i = pl.program_id(1)
    j = pl.program_id(2)

    p = p_ref[0]
    g = g_ref[0]
    bm = p.shape[0]
    bn = g.shape[1]

    px, py, pz = p[:, 0:1], p[:, 1:2], p[:, 2:3]
    gx, gy, gz = g[0:1, :], g[1:2, :], g[2:3, :]
    aa = px * px + py * py + pz * pz
    bb = gx * gx + gy * gy + gz * gz
    t = aa + bb

    ab2 = jax.lax.dot_general(
        p.astype(jnp.bfloat16), g.astype(jnp.bfloat16) * jnp.bfloat16(2.0),
        (((1,), (0,)), ((), ())), preferred_element_type=jnp.float32)
    approx = t - ab2

    inf = jnp.float32(jnp.inf)
    big = jnp.int32(0x3FFFFFFF)

    tmin = jnp.min(approx, axis=1, keepdims=True)
    li = lax.broadcasted_iota(jnp.int32, (bm, bn), 1)
    tidx = jnp.min(jnp.where(approx == tmin, li, big),
                   axis=1, keepdims=True) + j * bn
    pmin = jnp.where(j == 0, inf, row_best[...])
    upd = tmin < pmin
    row_best[...] = jnp.where(upd, tmin, pmin)
    row_idx[...] = jnp.where(upd, tidx, row_idx[...])

    csl = (slice(None), pl.ds(j * bn, bn))
    ctmin = jnp.min(approx, axis=0, keepdims=True)
    si = lax.broadcasted_iota(jnp.int32, (bm, bn), 0)
    ctidx = jnp.min(jnp.where(approx == ctmin, si, big),
                    axis=0, keepdims=True) + i * bm
    cpmin = jnp.where(i == 0, inf, col_best[csl])
    cupd = ctmin < cpmin
    col_best[csl] = jnp.where(cupd, ctmin, cpmin)
    col_idx[csl] = jnp.where(cupd, ctidx, col_idx[csl])

    @pl.when(j == nj - 1)
    def _():
        rowidx_ref[0] = row_idx[...]

    @pl.when(i == ni - 1)
    def _():
        colidx_ref[0] = col_idx[csl]


def _argmin_call(p_t, gt_pc, bm, bn):
    B, M, _ = p_t.shape
    N = gt_pc.shape[2]
    ni = M // bm
    nj = N // bn
    return pl.pallas_call(
        functools.partial(_argmin_kernel, ni=ni, nj=nj),
        grid=(B, ni, nj),
        in_specs=[
            pl.BlockSpec((1, bm, 3), lambda b, i, j: (b, i, 0)),
            pl.BlockSpec((1, 3, bn), lambda b, i, j: (b, 0, j)),
        ],
        out_specs=[
            pl.BlockSpec((1, bm, 1), lambda b, i, j: (b, i, 0)),
            pl.BlockSpec((1, 1, bn), lambda b, i, j: (b, 0, j)),
        ],
        out_shape=[
            jax.ShapeDtypeStruct((B, M, 1), jnp.int32),
            jax.ShapeDtypeStruct((B, 1, N), jnp.int32),
        ],
        scratch_shapes=[
            pltpu.VMEM((bm, 1), jnp.float32),
            pltpu.VMEM((bm, 1), jnp.int32),
            pltpu.VMEM((1, N), jnp.float32),
            pltpu.VMEM((1, N), jnp.int32),
        ],
    )(p_t, gt_pc)


_NC, _NS = 2, 16
_NW = _NC * _NS
_LANES = 16


def _sc_gather_kernel(pred_h, gt_h, ir_h, ic_h, d2f_h, d2b_h,
                      tx_v, ty_v, tz_v, idx_v, qx_v, qy_v, qz_v, o_v,
                      *, B, M, N):
    wid = lax.axis_index("s") * _NC + lax.axis_index("c")
    wpb = _NW // B
    batch = wid // wpb
    chunk_f = M // wpb
    chunk_b = N // wpb
    qoff_f = (wid % wpb) * chunk_f
    qoff_b = (wid % wpb) * chunk_b

    def one_direction(table_h, tsize, query_h, qsize, qoff, idx_h, out_h,
                      out_base, chunk):
        pltpu.sync_copy(table_h.at[pl.ds((batch * 3 + 0) * tsize, tsize)], tx_v)
        pltpu.sync_copy(table_h.at[pl.ds((batch * 3 + 1) * tsize, tsize)], ty_v)
        pltpu.sync_copy(table_h.at[pl.ds((batch * 3 + 2) * tsize, tsize)], tz_v)
        pltpu.sync_copy(query_h.at[pl.ds((batch * 3 + 0) * qsize + qoff, chunk)],
                        qx_v.at[pl.ds(0, chunk)])
        pltpu.sync_copy(query_h.at[pl.ds((batch * 3 + 1) * qsize + qoff, chunk)],
                        qy_v.at[pl.ds(0, chunk)])
        pltpu.sync_copy(query_h.at[pl.ds((batch * 3 + 2) * qsize + qoff, chunk)],
                        qz_v.at[pl.ds(0, chunk)])
        pltpu.sync_copy(idx_h.at[pl.ds(out_base, chunk)],
                        idx_v.at[pl.ds(0, chunk)])
        for q in range(chunk // _LANES):
            sl = pl.ds(q * _LANES, _LANES)
            iv = idx_v[sl]
            dx = qx_v[sl] - plsc.load_gather(tx_v, [iv])
            dy = qy_v[sl] - plsc.load_gather(ty_v, [iv])
            dz = qz_v[sl] - plsc.load_gather(tz_v, [iv])
            o_v[sl] = dx * dx + dy * dy + dz * dz
        pltpu.sync_copy(o_v.at[pl.ds(0, chunk)], out_h.at[pl.ds(out_base, chunk)])

    one_direction(gt_h, N, pred_h, M, qoff_f, ir_h, d2f_h,
                  batch * M + qoff_f, chunk_f)
    one_direction(pred_h, M, gt_h, N, qoff_b, ic_h, d2b_h,
                  batch * N + qoff_b, chunk_b)


def _sc_gather_call(pred_flat, gt_flat, ir, ic, B, M, N):
    tmax = max(M, N)
    cmax = max(M, N) // (_NW // B)
    return pl.kernel(
        functools.partial(_sc_gather_kernel, B=B, M=M, N=N),
        out_type=[jax.ShapeDtypeStruct((B * M,), jnp.float32),
                  jax.ShapeDtypeStruct((B * N,), jnp.float32)],
        mesh=plsc.VectorSubcoreMesh(core_axis_name="c", subcore_axis_name="s"),
        compiler_params=pltpu.CompilerParams(needs_layout_passes=False),
        scratch_types=[
            pltpu.VMEM((tmax,), jnp.float32),
            pltpu.VMEM((tmax,), jnp.float32),
            pltpu.VMEM((tmax,), jnp.float32),
            pltpu.VMEM((cmax,), jnp.int32),
            pltpu.VMEM((cmax,), jnp.float32),
            pltpu.VMEM((cmax,), jnp.float32),
            pltpu.VMEM((cmax,), jnp.float32),
            pltpu.VMEM((cmax,), jnp.float32),
        ],
    )(pred_flat, gt_flat, ir, ic)


def _reduce_kernel(d2f_ref, d2b_ref, out_ref, *, denom_m, denom_n):
    s_f = jnp.sum(jnp.sqrt(d2f_ref[...] + 1e-8))
    s_b = jnp.sum(jnp.sqrt(d2b_ref[...] + 1e-8))
    out_ref[...] = jnp.full((1, 1), s_f / denom_m + s_b / denom_n, jnp.float32)


def _reduce_call(d2f, d2b, denom_m, denom_n):
    rows_f = d2f.size // 128
    rows_b = d2b.size // 128
    return pl.pallas_call(
        functools.partial(_reduce_kernel, denom_m=denom_m, denom_n=denom_n),
        out_shape=jax.ShapeDtypeStruct((1, 1), jnp.float32),
    )(d2f.reshape(rows_f, 128), d2b.reshape(rows_b, 128))


@jax.jit
def kernel(predict_pc, gt_pc):
    B, _, M = predict_pc.shape
    N = gt_pc.shape[2]
    bm = min(512, M)
    bn = min(8192, N)
    p_t = jnp.swapaxes(predict_pc, 1, 2)
    idx_row, idx_col = _argmin_call(p_t, gt_pc, bm, bn)

    d2f, d2b = _sc_gather_call(
        predict_pc.reshape(B * 3 * M), gt_pc.reshape(B * 3 * N),
        idx_row.reshape(B * M), idx_col.reshape(B * N), B, M, N)

    out = _reduce_call(d2f, d2b, float(B * M), float(B * N))
    return out[0, 0]
